# baseline (speedup 1.0000x reference)
import functools

import jax
import jax.numpy as jnp
from jax import lax
from jax.experimental import pallas as pl
from jax.experimental.pallas import tpu as pltpu
from jax.experimental.pallas import tpu_sc as plsc

_CHUNK = 64
_DPAD = 128
_NBUF = 5


@functools.cache
def _make_pad(V: int, D: int):
    rows = 2048
    grid = (V + rows - 1) // rows

    def pad_block(x_ref, o_ref):
        o_ref[:, :D] = x_ref[...]
        o_ref[:, D:] = jnp.zeros_like(o_ref[:, D:])

    return pl.pallas_call(
        pad_block,
        grid=(grid,),
        in_specs=[pl.BlockSpec((rows, D), lambda i: (i, 0))],
        out_specs=pl.BlockSpec((rows, _DPAD), lambda i: (i, 0)),
        out_shape=jax.ShapeDtypeStruct((V, _DPAD), jnp.float32),
    )


@functools.cache
def _make_gather(B: int, D: int):
    info = plsc.get_sparse_core_info()
    nw = info.num_cores * info.num_subcores
    b_per_w = B // nw
    n_chunks = b_per_w // _CHUNK
    n_groups = n_chunks // _NBUF
    mesh = plsc.VectorSubcoreMesh(core_axis_name="c", subcore_axis_name="s")

    @functools.partial(
        pl.kernel,
        out_type=jax.ShapeDtypeStruct((B, D), jnp.float32),
        mesh=mesh,
        scratch_types=[
            pltpu.VMEM((b_per_w,), jnp.int32),
            [pltpu.VMEM((_CHUNK, _DPAD), jnp.float32)] * _NBUF,
            [pltpu.VMEM((_CHUNK, D), jnp.float32)] * _NBUF,
            [pltpu.SemaphoreType.DMA] * _NBUF,
            [pltpu.SemaphoreType.DMA] * _NBUF,
        ],
    )
    def gather_kernel(table_hbm, idx_hbm, out_hbm, idx_v, wide, packed,
                      gsems, wsems):
        wid = lax.axis_index("s") * info.num_cores + lax.axis_index("c")
        base = wid * b_per_w

        pltpu.sync_copy(idx_hbm.at[pl.ds(base, b_per_w)], idx_v)

        def gather_desc(c, b):
            return pltpu.make_async_copy(
                table_hbm.at[idx_v.at[pl.ds(c * _CHUNK, _CHUNK)]],
                wide[b],
                gsems[b],
            )

        def wait_write(b):
            pltpu.make_async_copy(
                packed[b], out_hbm.at[pl.ds(base, _CHUNK)], wsems[b]
            ).wait()

        def compact(b):
            def rows4(r4, carry):
                r = r4 * 4
                for dr in range(4):
                    for k in (0, 16, 32, 48, 64, 80, D - 16):
                        packed[b][r + dr, pl.ds(k, 16)] = (
                            wide[b][r + dr, pl.ds(k, 16)]
                        )
                return carry

            lax.fori_loop(0, _CHUNK // 4, rows4, 0)

        def body(g, carry):
            c0 = g * _NBUF
            for b in range(_NBUF):
                @pl.when(g > 0)
                def _():
                    wait_write(b)
                gather_desc(c0 + b, b).start()
            for b in range(_NBUF):
                gather_desc(c0 + b, b).wait()
                compact(b)
                off = base + (c0 + b) * _CHUNK
                pltpu.async_copy(
                    packed[b], out_hbm.at[pl.ds(off, _CHUNK)], wsems[b]
                )
            return carry

        lax.fori_loop(0, n_groups, body, 0)
        for b in range(_NBUF):
            wait_write(b)

    return gather_kernel


@functools.cache
def _make_format(S: int, T: int, D: int):
    G = 16

    def fmt_block(x_ref, o_ref):
        o_ref[...] = x_ref[...].reshape(G, T, D)

    return pl.pallas_call(
        fmt_block,
        grid=(S // G,),
        in_specs=[pl.BlockSpec((G * T, D), lambda i: (i, 0))],
        out_specs=pl.BlockSpec((G, T, D), lambda i: (i, 0, 0)),
        out_shape=jax.ShapeDtypeStruct((S, T, D), jnp.float32),
    )


def kernel(sequence, embedding_matrix):
    S, T = sequence.shape
    B = S * T
    V, D = embedding_matrix.shape
    idx = sequence.reshape(B).astype(jnp.int32)
    table_p = _make_pad(V, D)(embedding_matrix)
    out = _make_gather(B, D)(table_p, idx)
    return _make_format(S, T, D)(out)


# speedup vs baseline: 1.5658x; 1.5658x over previous
import functools

import jax
import jax.numpy as jnp
from jax import lax
from jax.experimental import pallas as pl
from jax.experimental.pallas import tpu as pltpu
from jax.experimental.pallas import tpu_sc as plsc

_CHUNK = 80
_DPAD = 128
_NBUF = 5


@functools.cache
def _make_pad(V: int, D: int):
    rows = 2048
    grid = (V + rows - 1) // rows

    def pad_block(x_ref, o_ref):
        o_ref[:, :D] = x_ref[...]
        o_ref[:, D:] = jnp.zeros_like(o_ref[:, D:])

    return pl.pallas_call(
        pad_block,
        grid=(grid,),
        in_specs=[pl.BlockSpec((rows, D), lambda i: (i, 0))],
        out_specs=pl.BlockSpec((rows, _DPAD), lambda i: (i, 0)),
        out_shape=jax.ShapeDtypeStruct((V, _DPAD), jnp.float32),
    )


@functools.cache
def _make_gather(B: int, D: int):
    info = plsc.get_sparse_core_info()
    nw = info.num_cores * info.num_subcores
    b_per_w = B // nw
    n_chunks = b_per_w // _CHUNK
    n_groups = n_chunks // _NBUF
    mesh = plsc.VectorSubcoreMesh(core_axis_name="c", subcore_axis_name="s")

    @functools.partial(
        pl.kernel,
        out_type=jax.ShapeDtypeStruct((B, D), jnp.float32),
        mesh=mesh,
        scratch_types=[
            pltpu.VMEM((b_per_w,), jnp.int32),
            [pltpu.VMEM((_CHUNK, _DPAD), jnp.float32)] * _NBUF,
            [pltpu.VMEM((_CHUNK, D), jnp.float32)] * _NBUF,
            [pltpu.SemaphoreType.DMA] * _NBUF,
            [pltpu.SemaphoreType.DMA] * _NBUF,
        ],
    )
    def gather_kernel(table_hbm, idx_hbm, out_hbm, idx_v, wide, packed,
                      gsems, wsems):
        wid = lax.axis_index("s") * info.num_cores + lax.axis_index("c")
        base = wid * b_per_w

        pltpu.sync_copy(idx_hbm.at[pl.ds(base, b_per_w)], idx_v)

        def gather_desc(c, b):
            return pltpu.make_async_copy(
                table_hbm.at[idx_v.at[pl.ds(c * _CHUNK, _CHUNK)]],
                wide[b],
                gsems[b],
            )

        def wait_write(b):
            pltpu.make_async_copy(
                packed[b], out_hbm.at[pl.ds(base, _CHUNK)], wsems[b]
            ).wait()

        def compact(b):
            def rows4(r4, carry):
                r = r4 * 4
                for dr in range(4):
                    for k in (0, 16, 32, 48, 64, 80, D - 16):
                        packed[b][r + dr, pl.ds(k, 16)] = (
                            wide[b][r + dr, pl.ds(k, 16)]
                        )
                return carry

            lax.fori_loop(0, _CHUNK // 4, rows4, 0)

        def body(g, carry):
            c0 = g * _NBUF
            for b in range(_NBUF):
                @pl.when(g > 0)
                def _():
                    wait_write(b)
                gather_desc(c0 + b, b).start()
            for b in range(_NBUF):
                gather_desc(c0 + b, b).wait()
                compact(b)
                off = base + (c0 + b) * _CHUNK
                pltpu.async_copy(
                    packed[b], out_hbm.at[pl.ds(off, _CHUNK)], wsems[b]
                )
            return carry

        lax.fori_loop(0, n_groups, body, 0)
        for b in range(_NBUF):
            wait_write(b)

    return gather_kernel


def kernel(sequence, embedding_matrix):
    seq_shape = sequence.shape
    B = seq_shape[0] * seq_shape[1]
    V, D = embedding_matrix.shape
    idx = sequence.reshape(B).astype(jnp.int32)
    table_p = _make_pad(V, D)(embedding_matrix)
    out = _make_gather(B, D)(table_p, idx)
    return out.reshape(seq_shape + (D,))


# speedup vs baseline: 1.5734x; 1.0049x over previous
import functools

import jax
import jax.numpy as jnp
from jax import lax
from jax.experimental import pallas as pl
from jax.experimental.pallas import tpu as pltpu
from jax.experimental.pallas import tpu_sc as plsc

_CHUNK = 64
_DPAD = 128
_NBUF = 5


@functools.cache
def _make_pad(V: int, D: int):
    rows = 2048
    grid = (V + rows - 1) // rows

    def pad_block(x_ref, o_ref):
        o_ref[:, :D] = x_ref[...]
        o_ref[:, D:] = jnp.zeros_like(o_ref[:, D:])

    return pl.pallas_call(
        pad_block,
        grid=(grid,),
        in_specs=[pl.BlockSpec((rows, D), lambda i: (i, 0))],
        out_specs=pl.BlockSpec((rows, _DPAD), lambda i: (i, 0)),
        out_shape=jax.ShapeDtypeStruct((V, _DPAD), jnp.float32),
    )


@functools.cache
def _make_gather(B: int, D: int):
    info = plsc.get_sparse_core_info()
    nw = info.num_cores * info.num_subcores
    b_per_w = B // nw
    n_chunks = b_per_w // _CHUNK
    n_groups = n_chunks // _NBUF
    mesh = plsc.VectorSubcoreMesh(core_axis_name="c", subcore_axis_name="s")

    @functools.partial(
        pl.kernel,
        out_type=jax.ShapeDtypeStruct((B, D), jnp.float32),
        mesh=mesh,
        scratch_types=[
            pltpu.VMEM((b_per_w,), jnp.int32),
            [pltpu.VMEM((_CHUNK, _DPAD), jnp.float32)] * _NBUF,
            [pltpu.VMEM((_CHUNK, D), jnp.float32)] * _NBUF,
            [pltpu.SemaphoreType.DMA] * _NBUF,
            [pltpu.SemaphoreType.DMA] * _NBUF,
        ],
    )
    def gather_kernel(table_hbm, idx_hbm, out_hbm, idx_v, wide, packed,
                      gsems, wsems):
        wid = lax.axis_index("s") * info.num_cores + lax.axis_index("c")
        base = wid * b_per_w

        pltpu.sync_copy(idx_hbm.at[pl.ds(base, b_per_w)], idx_v)

        def gather_desc(c, b):
            return pltpu.make_async_copy(
                table_hbm.at[idx_v.at[pl.ds(c * _CHUNK, _CHUNK)]],
                wide[b],
                gsems[b],
            )

        def wait_write(b):
            pltpu.make_async_copy(
                packed[b], out_hbm.at[pl.ds(base, _CHUNK)], wsems[b]
            ).wait()

        def compact(b):
            def rows4(r4, carry):
                r = r4 * 4
                for dr in range(4):
                    for k in (0, 16, 32, 48, 64, 80, D - 16):
                        packed[b][r + dr, pl.ds(k, 16)] = (
                            wide[b][r + dr, pl.ds(k, 16)]
                        )
                return carry

            lax.fori_loop(0, _CHUNK // 4, rows4, 0)

        def body(g, carry):
            c0 = g * _NBUF
            for b in range(_NBUF):
                @pl.when(g > 0)
                def _():
                    wait_write(b)
                gather_desc(c0 + b, b).start()
            for b in range(_NBUF):
                gather_desc(c0 + b, b).wait()
                compact(b)
                off = base + (c0 + b) * _CHUNK
                pltpu.async_copy(
                    packed[b], out_hbm.at[pl.ds(off, _CHUNK)], wsems[b]
                )
            return carry

        lax.fori_loop(0, n_groups, body, 0)
        for b in range(_NBUF):
            wait_write(b)

    return gather_kernel


def kernel(sequence, embedding_matrix):
    seq_shape = sequence.shape
    B = seq_shape[0] * seq_shape[1]
    V, D = embedding_matrix.shape
    idx = sequence.reshape(B).astype(jnp.int32)
    table_p = _make_pad(V, D)(embedding_matrix)
    out = _make_gather(B, D)(table_p, idx)
    return out.reshape(seq_shape + (D,))
